# Initial kernel scaffold; baseline (speedup 1.0000x reference)
import jax
import jax.numpy as jnp
from jax.experimental import pallas as pl


def kernel(u_t, W1, b1, W2, b2, Wr, br):
    raise NotImplementedError("write your pallas kernel here")


# speedup vs baseline: 1.6349x; 1.6349x over previous
import jax
import jax.numpy as jnp
from jax.experimental import pallas as pl
from jax.experimental.pallas import tpu as pltpu

B, S, C = 8, 2048, 64
P, TOPK, DFF = 8, 2, 128


def _moe_body(u_ref, w1_ref, b1_ref, w2_ref, b2_ref, wr_ref, br_ref, out_ref):
    u = u_ref[0]
    pooled = jnp.mean(u, axis=0, keepdims=True)
    logits = (
        jnp.dot(pooled, wr_ref[...], preferred_element_type=jnp.float32)
        + br_ref[...]
    )
    iota = jax.lax.broadcasted_iota(jnp.int32, (1, P), 1)
    v1 = jnp.max(logits)
    idx1 = jnp.argmax(logits)
    masked = jnp.where(iota == idx1, -jnp.inf, logits)
    v2 = jnp.max(masked)
    idx2 = jnp.argmax(masked)
    z = jnp.exp(v2 - v1)
    g1 = 1.0 / (1.0 + z)
    g2 = z / (1.0 + z)

    def expert_delta(e):
        w1 = w1_ref[e]
        w2 = w2_ref[e]
        bb1 = b1_ref[pl.ds(e, 1), :]
        bb2 = b2_ref[pl.ds(e, 1), :]
        h = jax.nn.gelu(
            jnp.dot(u, w1, preferred_element_type=jnp.float32) + bb1
        )
        return jnp.dot(h, w2, preferred_element_type=jnp.float32) + bb2

    out_ref[0] = u + g1 * expert_delta(idx1.astype(jnp.int32)) + g2 * expert_delta(
        idx2.astype(jnp.int32)
    )


def kernel(u_t, W1, b1, W2, b2, Wr, br):
    br2 = br.reshape(1, P)
    grid = (B,)
    return pl.pallas_call(
        _moe_body,
        grid=grid,
        in_specs=[
            pl.BlockSpec((1, S, C), lambda b: (b, 0, 0)),
            pl.BlockSpec((P, C, DFF), lambda b: (0, 0, 0)),
            pl.BlockSpec((P, DFF), lambda b: (0, 0)),
            pl.BlockSpec((P, DFF, C), lambda b: (0, 0, 0)),
            pl.BlockSpec((P, C), lambda b: (0, 0)),
            pl.BlockSpec((C, P), lambda b: (0, 0)),
            pl.BlockSpec((1, P), lambda b: (0, 0)),
        ],
        out_specs=pl.BlockSpec((1, S, C), lambda b: (b, 0, 0)),
        out_shape=jax.ShapeDtypeStruct((B, S, C), jnp.float32),
        compiler_params=pltpu.CompilerParams(
            dimension_semantics=("arbitrary",),
        ),
    )(u_t, W1, b1, W2, b2, Wr, br2)
